# baseline (speedup 1.0000x reference)
import functools

import jax
import jax.numpy as jnp
from jax import lax
from jax.experimental import pallas as pl
from jax.experimental.pallas import tpu as pltpu
from jax.experimental.pallas import tpu_sc as plsc

VOCAB = 1024
D_MODEL = 1024
SEQ_LEN = 2048
BATCH = 4
MAX_TOK = 20
SCALE = 8.0 ** 0.5

NC = 2
NS = 16
NW = NC * NS
LANES = 16

W_PER_B = NW // BATCH
D_PER_W = D_MODEL // W_PER_B
D_CHUNK = 16
N_CHUNK = D_PER_W // D_CHUNK
N_GRP = SEQ_LEN // LANES

TBL_ROWS = 24
TBL = TBL_ROWS * D_MODEL


def _sc_body(x_hbm, w_hbm, out_hbm, x_v, tbl_v, idx_v, out0_v, out1_v,
             grp_s, sem0, sem1):
    wid = lax.axis_index("s") * NC + lax.axis_index("c")
    b = wid // W_PER_B
    d0 = (wid % W_PER_B) * D_PER_W

    stage = [
        pltpu.async_copy(w_hbm.at[r],
                         tbl_v.at[pl.ds(r * D_MODEL, D_MODEL)], sem0)
        for r in range(MAX_TOK)
    ]
    stage.append(pltpu.async_copy(x_hbm, x_v, sem1))

    zvec = jnp.zeros((LANES,), jnp.float32)
    for buf in (out0_v, out1_v):
        @plsc.parallel_loop(0, D_CHUNK * N_GRP, unroll=8)
        def _zfill(i, buf=buf):
            dd = i // N_GRP
            g = i - dd * N_GRP
            buf[dd, pl.ds(g * LANES, LANES)] = zvec

    for cp in stage:
        cp.wait()

    @plsc.parallel_loop(0, MAX_TOK * (D_MODEL // LANES), unroll=4)
    def _scale(i):
        p = i * LANES
        tbl_v[pl.ds(p, LANES)] = tbl_v[pl.ds(p, LANES)] * SCALE

    @plsc.parallel_loop(0, D_MODEL // LANES, unroll=4)
    def _zero(j):
        tbl_v[pl.ds(MAX_TOK * D_MODEL + j * LANES, LANES)] = zvec

    lane_step = lax.iota(jnp.int32, LANES) * BATCH

    def _extract(g, ngrp):
        flat = g * (LANES * BATCH) + b + lane_step
        v = plsc.load_gather(x_v, [flat])
        slot = pl.multiple_of(ngrp * LANES, LANES)
        idx_v[pl.ds(slot, LANES)] = jnp.minimum(v, MAX_TOK) * D_MODEL + d0
        grp_s[ngrp] = g
        return ngrp + jnp.where(jnp.min(v) < MAX_TOK, 1, 0).astype(ngrp.dtype)
    ngrp = lax.fori_loop(0, N_GRP, _extract, jnp.int32(0))

    bufs = (out0_v, out1_v)
    sems = (sem0, sem1)
    copies = [None, None]
    for c in range(N_CHUNK):
        nbuf = c % 2
        if copies[nbuf] is not None:
            copies[nbuf].wait()

        out_v = bufs[nbuf]
        doffs = [jnp.full((LANES,), c * D_CHUNK + dd, jnp.int32)
                 for dd in range(D_CHUNK)]

        @plsc.parallel_loop(0, ngrp, unroll=4)
        def _live(j, out_v=out_v, doffs=doffs):
            slot = pl.multiple_of(j * LANES, LANES)
            base = pl.multiple_of(grp_s[j] * LANES, LANES)
            rows = idx_v[pl.ds(slot, LANES)]
            vals = [plsc.load_gather(tbl_v, [rows + doffs[dd]])
                    for dd in range(D_CHUNK)]
            for dd in range(D_CHUNK):
                out_v[dd, pl.ds(base, LANES)] = vals[dd]

        copies[nbuf] = pltpu.async_copy(
            out_v, out_hbm.at[b, pl.ds(d0 + c * D_CHUNK, D_CHUNK), :],
            sems[nbuf])

    copies[0].wait()
    copies[1].wait()


_sc_embed = functools.partial(
    pl.kernel,
    mesh=plsc.VectorSubcoreMesh(
        core_axis_name="c", subcore_axis_name="s",
        num_cores=NC, num_subcores=NS),
    out_type=jax.ShapeDtypeStruct((BATCH, D_MODEL, SEQ_LEN), jnp.float32),
    compiler_params=pltpu.CompilerParams(needs_layout_passes=False),
    scratch_types=[
        pltpu.VMEM((SEQ_LEN * BATCH,), jnp.int32),
        pltpu.VMEM((TBL,), jnp.float32),
        pltpu.VMEM((SEQ_LEN,), jnp.int32),
        pltpu.VMEM((D_CHUNK, SEQ_LEN), jnp.float32),
        pltpu.VMEM((D_CHUNK, SEQ_LEN), jnp.float32),
        pltpu.SMEM((N_GRP,), jnp.int32),
        pltpu.SemaphoreType.DMA,
        pltpu.SemaphoreType.DMA,
    ],
)(_sc_body)


def kernel(x, embed_w):
    return _sc_embed(x.astype(jnp.int32).reshape(-1), embed_w)


# speedup vs baseline: 1.0203x; 1.0203x over previous
import functools

import jax
import jax.numpy as jnp
from jax import lax
from jax.experimental import pallas as pl
from jax.experimental.pallas import tpu as pltpu
from jax.experimental.pallas import tpu_sc as plsc

VOCAB = 1024
D_MODEL = 1024
SEQ_LEN = 2048
BATCH = 4
MAX_TOK = 20
SCALE = 8.0 ** 0.5

NC = 2
NS = 16
NW = NC * NS
LANES = 16

W_PER_B = NW // BATCH
D_PER_W = D_MODEL // W_PER_B
D_CHUNK = 16
N_CHUNK = D_PER_W // D_CHUNK
N_GRP = SEQ_LEN // LANES

TBL_ROWS = 24
TBL = TBL_ROWS * D_MODEL


def _sc_body(x_hbm, w_hbm, out_hbm, x_v, tbl_v, idx_v, out0_v, out1_v,
             grp_s, sem0, sem1):
    wid = lax.axis_index("s") * NC + lax.axis_index("c")
    b = wid // W_PER_B
    d0 = (wid % W_PER_B) * D_PER_W

    stage = [
        pltpu.async_copy(w_hbm.at[r],
                         tbl_v.at[pl.ds(r * D_MODEL, D_MODEL)], sem0)
        for r in range(MAX_TOK)
    ]
    stage.append(pltpu.async_copy(x_hbm, x_v, sem1))

    zvec = jnp.zeros((LANES,), jnp.float32)
    for buf in (out0_v, out1_v):
        @plsc.parallel_loop(0, D_CHUNK * N_GRP, unroll=8)
        def _zfill(i, buf=buf):
            dd = i // N_GRP
            g = i - dd * N_GRP
            buf[dd, pl.ds(g * LANES, LANES)] = zvec

    for cp in stage:
        cp.wait()

    @plsc.parallel_loop(0, MAX_TOK * (D_MODEL // LANES), unroll=4)
    def _scale(i):
        p = i * LANES
        tbl_v[pl.ds(p, LANES)] = tbl_v[pl.ds(p, LANES)] * SCALE

    @plsc.parallel_loop(0, D_MODEL // LANES, unroll=4)
    def _zero(j):
        tbl_v[pl.ds(MAX_TOK * D_MODEL + j * LANES, LANES)] = zvec

    lane_step = lax.iota(jnp.int32, LANES) * BATCH

    def _extract(g, ngrp):
        flat = g * (LANES * BATCH) + b + lane_step
        v = plsc.load_gather(x_v, [flat])
        slot = pl.multiple_of(ngrp * LANES, LANES)
        idx_v[pl.ds(slot, LANES)] = jnp.minimum(v, MAX_TOK) * D_MODEL + d0
        grp_s[ngrp] = g
        return ngrp + jnp.where(jnp.min(v) < MAX_TOK, 1, 0).astype(ngrp.dtype)
    ngrp = lax.fori_loop(0, N_GRP, _extract, jnp.int32(0))

    bufs = (out0_v, out1_v)
    sems = (sem0, sem1)
    copies = [None, None]
    for c in range(N_CHUNK):
        nbuf = c % 2
        if copies[nbuf] is not None:
            copies[nbuf].wait()

        out_v = bufs[nbuf]
        doffs = [jnp.full((LANES,), c * D_CHUNK + dd, jnp.int32)
                 for dd in range(D_CHUNK)]

        @plsc.parallel_loop(0, ngrp, unroll=2)
        def _live(j, out_v=out_v, doffs=doffs):
            slot = pl.multiple_of(j * LANES, LANES)
            base = pl.multiple_of(grp_s[j] * LANES, LANES)
            rows = idx_v[pl.ds(slot, LANES)]
            vals = [plsc.load_gather(tbl_v, [rows + doffs[dd]])
                    for dd in range(D_CHUNK)]
            for dd in range(D_CHUNK):
                out_v[dd, pl.ds(base, LANES)] = vals[dd]

        copies[nbuf] = pltpu.async_copy(
            out_v, out_hbm.at[b, pl.ds(d0 + c * D_CHUNK, D_CHUNK), :],
            sems[nbuf])

    copies[0].wait()
    copies[1].wait()


_sc_embed = functools.partial(
    pl.kernel,
    mesh=plsc.VectorSubcoreMesh(
        core_axis_name="c", subcore_axis_name="s",
        num_cores=NC, num_subcores=NS),
    out_type=jax.ShapeDtypeStruct((BATCH, D_MODEL, SEQ_LEN), jnp.float32),
    compiler_params=pltpu.CompilerParams(needs_layout_passes=False),
    scratch_types=[
        pltpu.VMEM((SEQ_LEN * BATCH,), jnp.int32),
        pltpu.VMEM((TBL,), jnp.float32),
        pltpu.VMEM((SEQ_LEN,), jnp.int32),
        pltpu.VMEM((D_CHUNK, SEQ_LEN), jnp.float32),
        pltpu.VMEM((D_CHUNK, SEQ_LEN), jnp.float32),
        pltpu.SMEM((N_GRP,), jnp.int32),
        pltpu.SemaphoreType.DMA,
        pltpu.SemaphoreType.DMA,
    ],
)(_sc_body)


def kernel(x, embed_w):
    return _sc_embed(x.astype(jnp.int32).reshape(-1), embed_w)


# speedup vs baseline: 1.0319x; 1.0114x over previous
import functools

import jax
import jax.numpy as jnp
from jax import lax
from jax.experimental import pallas as pl
from jax.experimental.pallas import tpu as pltpu
from jax.experimental.pallas import tpu_sc as plsc

VOCAB = 1024
D_MODEL = 1024
SEQ_LEN = 2048
BATCH = 4
MAX_TOK = 20
SCALE = 8.0 ** 0.5

NC = 2
NS = 16
NW = NC * NS
LANES = 16

W_PER_B = NW // BATCH
D_PER_W = D_MODEL // W_PER_B
D_CHUNK = 16
N_CHUNK = D_PER_W // D_CHUNK
N_GRP = SEQ_LEN // LANES

TBL_ROWS = 24
TBL = TBL_ROWS * D_MODEL


def _sc_body(x_hbm, w_hbm, out_hbm, x_v, tbl_v, idx_v, out0_v, out1_v,
             grp_s, sem0, sem1):
    wid = lax.axis_index("s") * NC + lax.axis_index("c")
    b = wid // W_PER_B
    d0 = (wid % W_PER_B) * D_PER_W

    stage = [
        pltpu.async_copy(w_hbm.at[r],
                         tbl_v.at[pl.ds(r * D_MODEL, D_MODEL)], sem0)
        for r in range(MAX_TOK)
    ]
    xcp = pltpu.async_copy(x_hbm, x_v, sem1)

    zvec = jnp.zeros((LANES,), jnp.float32)
    for buf in (out0_v, out1_v):
        @plsc.parallel_loop(0, D_CHUNK * N_GRP, unroll=8)
        def _zfill(i, buf=buf):
            dd = i // N_GRP
            g = i - dd * N_GRP
            buf[dd, pl.ds(g * LANES, LANES)] = zvec

    xcp.wait()
    lane_step = lax.iota(jnp.int32, LANES) * BATCH

    def _extract(g, ngrp):
        flat = g * (LANES * BATCH) + b + lane_step
        v = plsc.load_gather(x_v, [flat])
        slot = pl.multiple_of(ngrp * LANES, LANES)
        idx_v[pl.ds(slot, LANES)] = jnp.minimum(v, MAX_TOK) * D_MODEL + d0
        grp_s[ngrp] = g
        return ngrp + jnp.where(jnp.min(v) < MAX_TOK, 1, 0).astype(ngrp.dtype)
    ngrp = lax.fori_loop(0, N_GRP, _extract, jnp.int32(0))

    for cp in stage:
        cp.wait()

    @plsc.parallel_loop(0, MAX_TOK * (D_MODEL // LANES), unroll=4)
    def _scale(i):
        p = i * LANES
        tbl_v[pl.ds(p, LANES)] = tbl_v[pl.ds(p, LANES)] * SCALE

    @plsc.parallel_loop(0, D_MODEL // LANES, unroll=4)
    def _zero(j):
        tbl_v[pl.ds(MAX_TOK * D_MODEL + j * LANES, LANES)] = zvec

    bufs = (out0_v, out1_v)
    sems = (sem0, sem1)
    copies = [None, None]
    for c in range(N_CHUNK):
        nbuf = c % 2
        if copies[nbuf] is not None:
            copies[nbuf].wait()

        out_v = bufs[nbuf]
        doffs = [jnp.full((LANES,), c * D_CHUNK + dd, jnp.int32)
                 for dd in range(D_CHUNK)]

        @plsc.parallel_loop(0, ngrp, unroll=2)
        def _live(j, out_v=out_v, doffs=doffs):
            slot = pl.multiple_of(j * LANES, LANES)
            base = pl.multiple_of(grp_s[j] * LANES, LANES)
            rows = idx_v[pl.ds(slot, LANES)]
            vals = [plsc.load_gather(tbl_v, [rows + doffs[dd]])
                    for dd in range(D_CHUNK)]
            for dd in range(D_CHUNK):
                out_v[dd, pl.ds(base, LANES)] = vals[dd]

        copies[nbuf] = pltpu.async_copy(
            out_v, out_hbm.at[b, pl.ds(d0 + c * D_CHUNK, D_CHUNK), :],
            sems[nbuf])

    copies[0].wait()
    copies[1].wait()


_sc_embed = functools.partial(
    pl.kernel,
    mesh=plsc.VectorSubcoreMesh(
        core_axis_name="c", subcore_axis_name="s",
        num_cores=NC, num_subcores=NS),
    out_type=jax.ShapeDtypeStruct((BATCH, D_MODEL, SEQ_LEN), jnp.float32),
    compiler_params=pltpu.CompilerParams(needs_layout_passes=False),
    scratch_types=[
        pltpu.VMEM((SEQ_LEN * BATCH,), jnp.int32),
        pltpu.VMEM((TBL,), jnp.float32),
        pltpu.VMEM((SEQ_LEN,), jnp.int32),
        pltpu.VMEM((D_CHUNK, SEQ_LEN), jnp.float32),
        pltpu.VMEM((D_CHUNK, SEQ_LEN), jnp.float32),
        pltpu.SMEM((N_GRP,), jnp.int32),
        pltpu.SemaphoreType.DMA,
        pltpu.SemaphoreType.DMA,
    ],
)(_sc_body)


def kernel(x, embed_w):
    return _sc_embed(x.astype(jnp.int32).reshape(-1), embed_w)
